# baseline (speedup 1.0000x reference)
import functools

import jax
import jax.numpy as jnp
from jax import lax
from jax.experimental import pallas as pl
from jax.experimental.pallas import tpu as pltpu
from jax.experimental.pallas import tpu_sc as plsc

N = 10000
E = 320000
H = 64
L = 28

_NC = 2
_NS = 16
_NW = _NC * _NS
_EPW = E // _NW
_CHUNK = 40
_NCHUNK = _EPW // _CHUNK
_RPT = 624


def _ln(h, g, b, eps=1e-5):
    mu = jnp.mean(h, axis=-1, keepdims=True)
    var = jnp.mean((h - mu) ** 2, axis=-1, keepdims=True)
    return (h - mu) / jnp.sqrt(var + eps) * g + b


def _sc_edge_kernel(tbl_hbm, idx2_hbm, mt_hbm, tv_hbm, out_hbm,
                    idx2_loc, rows, rows2, accst, bstg, mloc, tloc,
                    zbuf, table, sem, sem2):
    c = lax.axis_index("c")
    s = lax.axis_index("s")
    wid = c * _NS + s
    ebase = wid * _EPW

    rbase = s * _RPT
    nrows = jnp.where(s == _NS - 1, N - (_NS - 1) * _RPT, _RPT)

    def _z(i, _):
        for k in range(8):
            zbuf.at[i][pl.ds(k * 16, 16)] = jnp.zeros((16,), jnp.float32)
        return 0
    lax.fori_loop(0, 16, _z, 0)

    def _zcp(i, _):
        pltpu.sync_copy(zbuf, table.at[pl.ds(rbase + i * 16, 16)])
        return 0
    lax.fori_loop(0, nrows // 16, _zcp, 0)

    pltpu.sync_copy(idx2_hbm.at[pl.ds(2 * ebase, 2 * _EPW)],
                    idx2_loc.at[pl.ds(0, 2 * _EPW)])
    pltpu.sync_copy(mt_hbm, mloc)
    pltpu.sync_copy(tv_hbm, tloc)

    plsc.subcore_barrier()

    tv = tloc[pl.ds(0, 16)]
    mv = [mloc[pl.ds(16 * k, 16)] for k in range(4)]

    z16 = jnp.zeros((16,), jnp.float32)

    def flush(cur, fd, accs):
        for k in range(8):
            accst.at[0][pl.ds(16 * k, 16)] = accs[k]

        @pl.when(fd == 0)
        def _():
            pltpu.sync_copy(accst, table.at[pl.ds(N + 2 * s, 1)])
            idf = lax.broadcast(cur, (16,)).astype(jnp.float32)
            for k in range(8):
                bstg.at[0][pl.ds(16 * k, 16)] = idf
            pltpu.sync_copy(bstg, table.at[pl.ds(N + 2 * s + 1, 1)])

        @pl.when(fd == 1)
        def _():
            pltpu.sync_copy(accst, table.at[pl.ds(cur, 1)])

    def make_edge_body(buf, ci):
        def edge_body(e, carry):
            cur, fd = carry[0], carry[1]
            accs = carry[2:]
            d = idx2_loc[pl.ds(2 * (ci * _CHUNK + e) + 1, 16)][0]
            ended = (d != cur) & (cur >= 0)

            @pl.when(ended)
            def _():
                flush(cur, fd, accs)

            out = [d, fd | ended.astype(jnp.int32)]
            for k in range(4):
                qn = (buf.at[2 * e][pl.ds(16 * k, 16)]
                      + buf.at[2 * e + 1][pl.ds(64 + 16 * k, 16)])
                m = jnp.maximum(qn, 0.0) + 1e-7
                w = jnp.exp(m * tv - mv[k])
                a1 = jnp.where(ended, 0.0, accs[k])
                a2 = jnp.where(ended, 0.0, accs[4 + k])
                out.append(a1 + w)
                out.append(a2 + m * w)
            return (out[0], out[1], out[2], out[4], out[6], out[8],
                    out[3], out[5], out[7], out[9])
        return edge_body

    def issue(ci, buf, sm):
        pltpu.async_copy(
            tbl_hbm.at[idx2_loc.at[pl.ds(ci * 2 * _CHUNK, 2 * _CHUNK)]],
            buf, sm)

    def wait_rows(buf, sm):
        pltpu.make_async_copy(
            tbl_hbm.at[idx2_loc.at[pl.ds(0, 2 * _CHUNK)]], buf, sm).wait()

    issue(0, rows, sem)

    def pair_body(i, carry):
        issue(2 * i + 1, rows2, sem2)
        wait_rows(rows, sem)
        carry = lax.fori_loop(0, _CHUNK, make_edge_body(rows, 2 * i), carry, unroll=2)

        @pl.when(i < _NCHUNK // 2 - 1)
        def _():
            issue(2 * i + 2, rows, sem)

        wait_rows(rows2, sem2)
        carry = lax.fori_loop(0, _CHUNK, make_edge_body(rows2, 2 * i + 1),
                              carry, unroll=2)
        return carry

    init = (jnp.int32(-1), jnp.int32(0)) + (z16,) * 8
    fin = lax.fori_loop(0, _NCHUNK // 2, pair_body, init)

    flush(fin[0], fin[1], fin[2:])

    plsc.subcore_barrier()

    @pl.when(s < _NS - 1)
    def _():
        pltpu.sync_copy(table.at[pl.ds(rbase, _RPT)],
                        out_hbm.at[c, pl.ds(rbase, _RPT)])

    @pl.when(s == _NS - 1)
    def _():
        last = (_NS - 1) * _RPT
        pltpu.sync_copy(table.at[pl.ds(last, N + 2 * _NS - last)],
                        out_hbm.at[c, pl.ds(last, N + 2 * _NS - last)])


@functools.partial(jax.jit, static_argnames=())
def _sc_edges(tbl, idx2, mt, tv):
    mesh = plsc.VectorSubcoreMesh(core_axis_name="c", subcore_axis_name="s")
    f = pl.kernel(
        _sc_edge_kernel,
        out_type=jax.ShapeDtypeStruct((_NC, N + 2 * _NS, 128), jnp.float32),
        mesh=mesh,
        scratch_types=[
            pltpu.VMEM((2 * _EPW + 16,), jnp.int32),
            pltpu.VMEM((2 * _CHUNK, 128), jnp.float32),
            pltpu.VMEM((2 * _CHUNK, 128), jnp.float32),
            pltpu.VMEM((1, 128), jnp.float32),
            pltpu.VMEM((1, 128), jnp.float32),
            pltpu.VMEM((H,), jnp.float32),
            pltpu.VMEM((16,), jnp.float32),
            pltpu.VMEM((16, 128), jnp.float32),
            pltpu.VMEM_SHARED((N + 2 * _NS, 128), jnp.float32),
            pltpu.SemaphoreType.DMA,
            pltpu.SemaphoreType.DMA,
        ],
    )
    return f(tbl, idx2, mt, tv)


_B = 2000
_G = N // _B


def _colmax_update(acc, local, g):
    @pl.when(g == 0)
    def _():
        acc[...] = jnp.full((1, H), -1e30, jnp.float32)
    acc[...] = jnp.maximum(acc[...], local)


def _enc_body(aux, xin, Wn, bn, We, be, h0, q0, p, npd, mt, cnpd, accq, accn):
    g = pl.program_id(0)
    xv = xin[...]
    pv = xv @ We[...]
    h = xv @ Wn[...] + bn[...]
    npdv = be[...] - pv
    qv = h + pv
    h0[...] = h
    q0[...] = qv
    p[...] = pv
    npd[...] = npdv
    _colmax_update(accq, jnp.max(qv, axis=0, keepdims=True), g)
    _colmax_update(accn, jnp.max(npdv, axis=0, keepdims=True), g)

    @pl.when(g == _G - 1)
    def _():
        t0 = aux[0]
        cn = accn[...]
        cnpd[...] = cn
        bmsg = jnp.maximum(accq[...] + cn, 0.0) + 1e-7
        mt[...] = jnp.maximum(t0 * bmsg, t0 * 1e-7)


def _encoder(xin, Wn, bn, We, be, t0):
    fullspec = lambda: pl.BlockSpec((None,) * 0)
    out = pl.pallas_call(
        _enc_body,
        grid=(_G,),
        in_specs=[
            pl.BlockSpec(memory_space=pltpu.SMEM),
            pl.BlockSpec((_B, 128), lambda g: (g, 0)),
            pl.BlockSpec((128, H), lambda g: (0, 0)),
            pl.BlockSpec((1, H), lambda g: (0, 0)),
            pl.BlockSpec((128, H), lambda g: (0, 0)),
            pl.BlockSpec((1, H), lambda g: (0, 0)),
        ],
        out_specs=[
            pl.BlockSpec((_B, H), lambda g: (g, 0)),
            pl.BlockSpec((_B, H), lambda g: (g, 0)),
            pl.BlockSpec((_B, H), lambda g: (g, 0)),
            pl.BlockSpec((_B, H), lambda g: (g, 0)),
            pl.BlockSpec((1, H), lambda g: (0, 0)),
            pl.BlockSpec((1, H), lambda g: (0, 0)),
        ],
        out_shape=[
            jax.ShapeDtypeStruct((N, H), jnp.float32),
            jax.ShapeDtypeStruct((N, H), jnp.float32),
            jax.ShapeDtypeStruct((N, H), jnp.float32),
            jax.ShapeDtypeStruct((N, H), jnp.float32),
            jax.ShapeDtypeStruct((1, H), jnp.float32),
            jax.ShapeDtypeStruct((1, H), jnp.float32),
        ],
        scratch_shapes=[pltpu.VMEM((1, H), jnp.float32),
                        pltpu.VMEM((1, H), jnp.float32)],
    )(t0.reshape(1), xin, Wn, bn.reshape(1, H), We, be.reshape(1, H))
    return out


def _conv_tail(aux, Sn, bnd, z, row0, W1i, b1i, g1i, be1i, W2i, b2i):
    St = Sn[0] + Sn[1]
    bv = jnp.concatenate([bnd[0], bnd[1]], axis=0).reshape(2 * _NS, 2, 128)
    bacc = bv[:, 0, :]
    bids = bv[:, 1, 0:1].astype(jnp.int32)
    onehot = (bids == row0 +
              lax.broadcasted_iota(jnp.int32, (2 * _NS, _B), 1)
              ).astype(jnp.float32)
    St = St + lax.dot_general(onehot, bacc, (((0,), (0,)), ((), ())),
                              preferred_element_type=jnp.float32)
    s1 = St[:, :H]
    s2 = St[:, H:]
    agg = s2 / (s1 + 1e-30)
    nrm = jnp.maximum(
        jnp.sqrt(jnp.sum(agg * agg, axis=-1, keepdims=True)), 1e-12)
    xn = jnp.sqrt(jnp.sum(z * z, axis=-1, keepdims=True))
    out = (agg / nrm) * xn * aux[0] + z
    hmid = _ln(out @ W1i + b1i, g1i, be1i)
    return jax.nn.relu(hmid) @ W2i + b2i


def _mid_body(aux, h, z, Sn, bnd, p, cnpd, W1i, b1i, g1i, be1i, W2i, b2i,
              glnn, blnn, hn, zn, qn, mtn, accq, *, first):
    g = pl.program_id(0)
    hd = _conv_tail(aux, Sn[...], bnd[...], z[...], g * _B, W1i[...],
                    b1i[...], g1i[...], be1i[...], W2i[...], b2i[...])
    hnew = hd if first else h[...] + hd
    znew = jax.nn.relu(_ln(hnew, glnn[...], blnn[...]))
    qnew = znew + p[...]
    hn[...] = hnew
    zn[...] = znew
    qn[...] = qnew
    _colmax_update(accq, jnp.max(qnew, axis=0, keepdims=True), g)

    @pl.when(g == _G - 1)
    def _():
        tn = aux[1]
        bmsg = jnp.maximum(accq[...] + cnpd[...], 0.0) + 1e-7
        mtn[...] = jnp.maximum(tn * bmsg, tn * 1e-7)


_vec = lambda r: pl.BlockSpec((1, r), lambda g: (0, 0))


def _mid(h, z, S, p, cnpd, Wb, glnn, blnn, scale_i, t_next, first):
    W1i, b1i, g1i, be1i, W2i, b2i = Wb
    aux = jnp.stack([scale_i, t_next])
    nspec = lambda: pl.BlockSpec((_B, H), lambda g: (g, 0))
    out = pl.pallas_call(
        functools.partial(_mid_body, first=first),
        grid=(_G,),
        in_specs=[
            pl.BlockSpec(memory_space=pltpu.SMEM),
            nspec(), nspec(),
            pl.BlockSpec((_NC, _B, 128), lambda g: (0, g, 0)),
            pl.BlockSpec((_NC, 2 * _NS, 128), lambda g: (0, 0, 0)),
            nspec(), _vec(H),
            pl.BlockSpec((H, 2 * H), lambda g: (0, 0)), _vec(2 * H),
            _vec(2 * H), _vec(2 * H),
            pl.BlockSpec((2 * H, H), lambda g: (0, 0)), _vec(H),
            _vec(H), _vec(H),
        ],
        out_specs=[nspec(), nspec(), nspec(), _vec(H)],
        out_shape=[
            jax.ShapeDtypeStruct((N, H), jnp.float32),
            jax.ShapeDtypeStruct((N, H), jnp.float32),
            jax.ShapeDtypeStruct((N, H), jnp.float32),
            jax.ShapeDtypeStruct((1, H), jnp.float32),
        ],
        scratch_shapes=[pltpu.VMEM((1, H), jnp.float32)],
    )(aux, h, z, S[:, :N], S[:, N:], p, cnpd, W1i, b1i.reshape(1, -1),
      g1i.reshape(1, -1), be1i.reshape(1, -1), W2i, b2i.reshape(1, -1),
      glnn.reshape(1, H), blnn.reshape(1, H))
    return out


def _last_body(aux, h, z, Sn, bnd, W1i, b1i, g1i, be1i, W2i, b2i, gln0,
               bln0, Wlin, blin, y):
    g = pl.program_id(0)
    hd = _conv_tail(aux, Sn[...], bnd[...], z[...], g * _B, W1i[...],
                    b1i[...], g1i[...], be1i[...], W2i[...], b2i[...])
    hnew = h[...] + hd
    hf = jax.nn.relu(_ln(hnew, gln0[...], bln0[...]))
    y[...] = hf @ Wlin[...] + blin[...]


def _last(h, z, S, Wb, gln0, bln0, Wlin, blin, scale_i):
    W1i, b1i, g1i, be1i, W2i, b2i = Wb
    aux = jnp.stack([scale_i, scale_i])
    nspec = lambda: pl.BlockSpec((_B, H), lambda g: (g, 0))
    OUT = Wlin.shape[1]
    return pl.pallas_call(
        _last_body,
        grid=(_G,),
        in_specs=[
            pl.BlockSpec(memory_space=pltpu.SMEM),
            nspec(), nspec(),
            pl.BlockSpec((_NC, _B, 128), lambda g: (0, g, 0)),
            pl.BlockSpec((_NC, 2 * _NS, 128), lambda g: (0, 0, 0)),
            pl.BlockSpec((H, 2 * H), lambda g: (0, 0)), _vec(2 * H),
            _vec(2 * H), _vec(2 * H),
            pl.BlockSpec((2 * H, H), lambda g: (0, 0)), _vec(H),
            _vec(H), _vec(H),
            pl.BlockSpec((H, OUT), lambda g: (0, 0)), _vec(OUT),
        ],
        out_specs=pl.BlockSpec((_B, OUT), lambda g: (g, 0)),
        out_shape=jax.ShapeDtypeStruct((N, OUT), jnp.float32),
    )(aux, h, z, S[:, :N], S[:, N:], W1i, b1i.reshape(1, -1),
      g1i.reshape(1, -1), be1i.reshape(1, -1), W2i, b2i.reshape(1, -1),
      gln0.reshape(1, H), bln0.reshape(1, H), Wlin, blin.reshape(1, -1))


def kernel(x, pos, edge_index, Wn, bn, We, be, t, scale, W1, b1, g1, be1,
           W2, b2, gln, bln, Wlin, blin):
    xin = jnp.concatenate([x, jax.lax.stop_gradient(pos)], axis=-1)
    src = edge_index[1]
    dst = edge_index[0]
    perm = jnp.argsort(dst)
    ssrc = src[perm]
    sdst = dst[perm]
    idx2 = jnp.stack([ssrc, sdst], axis=1).reshape(-1)

    h, q, p, npd, mt, cnpd = _encoder(xin, Wn, bn, We, be, t[0])
    z = h

    for i in range(L):
        tbl = jnp.concatenate([q, npd], axis=1)
        tv = jnp.broadcast_to(t[i], (16,)).astype(jnp.float32)
        S = _sc_edges(tbl, idx2, mt.reshape(H), tv)
        Wb = (W1[i], b1[i], g1[i], be1[i], W2[i], b2[i])
        if i < L - 1:
            h, z, q, mt = _mid(h, z, S, p, cnpd, Wb, gln[i + 1], bln[i + 1],
                               scale[i], t[i + 1], first=(i == 0))
        else:
            y = _last(h, z, S, Wb, gln[0], bln[0], Wlin, blin, scale[i])
    return y


# speedup vs baseline: 1.0097x; 1.0097x over previous
import functools

import jax
import jax.numpy as jnp
from jax import lax
from jax.experimental import pallas as pl
from jax.experimental.pallas import tpu as pltpu
from jax.experimental.pallas import tpu_sc as plsc

N = 10000
E = 320000
H = 64
L = 28

_NC = 2
_NS = 16
_NW = _NC * _NS
_EPW = E // _NW
_CHUNK = 40
_NCHUNK = _EPW // _CHUNK
_RPT = 624


def _ln(h, g, b, eps=1e-5):
    mu = jnp.mean(h, axis=-1, keepdims=True)
    var = jnp.mean((h - mu) ** 2, axis=-1, keepdims=True)
    return (h - mu) / jnp.sqrt(var + eps) * g + b


def _sc_edge_kernel(tbl_hbm, idx2_hbm, mt_hbm, tv_hbm, out_hbm,
                    idx2_loc, rows, rows2, accst, bstg, mloc, tloc,
                    zbuf, table, sem, sem2):
    c = lax.axis_index("c")
    s = lax.axis_index("s")
    wid = c * _NS + s
    ebase = wid * _EPW

    rbase = s * _RPT
    nrows = jnp.where(s == _NS - 1, N - (_NS - 1) * _RPT, _RPT)

    def _z(i, _):
        for k in range(8):
            zbuf.at[i][pl.ds(k * 16, 16)] = jnp.zeros((16,), jnp.float32)
        return 0
    lax.fori_loop(0, 16, _z, 0)

    def _zcp(i, _):
        pltpu.sync_copy(zbuf, table.at[pl.ds(rbase + i * 16, 16)])
        return 0
    lax.fori_loop(0, nrows // 16, _zcp, 0)

    pltpu.sync_copy(idx2_hbm.at[pl.ds(2 * ebase, 2 * _EPW)],
                    idx2_loc.at[pl.ds(0, 2 * _EPW)])
    pltpu.sync_copy(mt_hbm, mloc)
    pltpu.sync_copy(tv_hbm, tloc)

    plsc.subcore_barrier()

    tv = tloc[pl.ds(0, 16)]
    mv = [mloc[pl.ds(16 * k, 16)] for k in range(4)]

    z16 = jnp.zeros((16,), jnp.float32)

    def flush(cur, fd, accs):
        for k in range(8):
            accst.at[0][pl.ds(16 * k, 16)] = accs[k]

        @pl.when(fd == 0)
        def _():
            pltpu.sync_copy(accst, table.at[pl.ds(N + 2 * s, 1)])
            idf = lax.broadcast(cur, (16,)).astype(jnp.float32)
            for k in range(8):
                bstg.at[0][pl.ds(16 * k, 16)] = idf
            pltpu.sync_copy(bstg, table.at[pl.ds(N + 2 * s + 1, 1)])

        @pl.when(fd == 1)
        def _():
            pltpu.sync_copy(accst, table.at[pl.ds(cur, 1)])

    def make_edge_body(buf, ci):
        def edge_body(e, carry):
            cur, fd = carry[0], carry[1]
            accs = carry[2:]
            d = idx2_loc[pl.ds(2 * (ci * _CHUNK + e) + 1, 16)][0]
            ended = (d != cur) & (cur >= 0)

            @pl.when(ended)
            def _():
                flush(cur, fd, accs)

            out = [d, fd | ended.astype(jnp.int32)]
            for k in range(4):
                qn = (buf.at[2 * e][pl.ds(16 * k, 16)]
                      + buf.at[2 * e + 1][pl.ds(64 + 16 * k, 16)])
                m = jnp.maximum(qn, 0.0) + 1e-7
                w = jnp.exp(m * tv - mv[k])
                a1 = jnp.where(ended, 0.0, accs[k])
                a2 = jnp.where(ended, 0.0, accs[4 + k])
                out.append(a1 + w)
                out.append(a2 + m * w)
            return (out[0], out[1], out[2], out[4], out[6], out[8],
                    out[3], out[5], out[7], out[9])
        return edge_body

    def issue(ci, buf, sm):
        pltpu.async_copy(
            tbl_hbm.at[idx2_loc.at[pl.ds(ci * 2 * _CHUNK, 2 * _CHUNK)]],
            buf, sm)

    def wait_rows(buf, sm):
        pltpu.make_async_copy(
            tbl_hbm.at[idx2_loc.at[pl.ds(0, 2 * _CHUNK)]], buf, sm).wait()

    issue(0, rows, sem)

    def pair_body(i, carry):
        issue(2 * i + 1, rows2, sem2)
        wait_rows(rows, sem)
        carry = lax.fori_loop(0, _CHUNK, make_edge_body(rows, 2 * i), carry)

        @pl.when(i < _NCHUNK // 2 - 1)
        def _():
            issue(2 * i + 2, rows, sem)

        wait_rows(rows2, sem2)
        carry = lax.fori_loop(0, _CHUNK, make_edge_body(rows2, 2 * i + 1),
                              carry)
        return carry

    init = (jnp.int32(-1), jnp.int32(0)) + (z16,) * 8
    fin = lax.fori_loop(0, _NCHUNK // 2, pair_body, init)

    flush(fin[0], fin[1], fin[2:])

    plsc.subcore_barrier()

    @pl.when(s < _NS - 1)
    def _():
        pltpu.sync_copy(table.at[pl.ds(rbase, _RPT)],
                        out_hbm.at[c, pl.ds(rbase, _RPT)])

    @pl.when(s == _NS - 1)
    def _():
        last = (_NS - 1) * _RPT
        pltpu.sync_copy(table.at[pl.ds(last, N + 2 * _NS - last)],
                        out_hbm.at[c, pl.ds(last, N + 2 * _NS - last)])


@functools.partial(jax.jit, static_argnames=())
def _sc_edges(tbl, idx2, mt, tv):
    mesh = plsc.VectorSubcoreMesh(core_axis_name="c", subcore_axis_name="s")
    f = pl.kernel(
        _sc_edge_kernel,
        out_type=jax.ShapeDtypeStruct((_NC, N + 2 * _NS, 128), jnp.float32),
        mesh=mesh,
        scratch_types=[
            pltpu.VMEM((2 * _EPW + 16,), jnp.int32),
            pltpu.VMEM((2 * _CHUNK, 128), jnp.float32),
            pltpu.VMEM((2 * _CHUNK, 128), jnp.float32),
            pltpu.VMEM((1, 128), jnp.float32),
            pltpu.VMEM((1, 128), jnp.float32),
            pltpu.VMEM((H,), jnp.float32),
            pltpu.VMEM((16,), jnp.float32),
            pltpu.VMEM((16, 128), jnp.float32),
            pltpu.VMEM_SHARED((N + 2 * _NS, 128), jnp.float32),
            pltpu.SemaphoreType.DMA,
            pltpu.SemaphoreType.DMA,
        ],
    )
    return f(tbl, idx2, mt, tv)


_B = 2000
_G = N // _B


def _colmax_update(acc, local, g):
    @pl.when(g == 0)
    def _():
        acc[...] = jnp.full((1, H), -1e30, jnp.float32)
    acc[...] = jnp.maximum(acc[...], local)


def _enc_body(aux, xin, Wn, bn, We, be, h0, q0, p, npd, mt, cnpd, accq, accn):
    g = pl.program_id(0)
    xv = xin[...]
    pv = xv @ We[...]
    h = xv @ Wn[...] + bn[...]
    npdv = be[...] - pv
    qv = h + pv
    h0[...] = h
    q0[...] = qv
    p[...] = pv
    npd[...] = npdv
    _colmax_update(accq, jnp.max(qv, axis=0, keepdims=True), g)
    _colmax_update(accn, jnp.max(npdv, axis=0, keepdims=True), g)

    @pl.when(g == _G - 1)
    def _():
        t0 = aux[0]
        cn = accn[...]
        cnpd[...] = cn
        bmsg = jnp.maximum(accq[...] + cn, 0.0) + 1e-7
        mt[...] = jnp.maximum(t0 * bmsg, t0 * 1e-7)


def _encoder(xin, Wn, bn, We, be, t0):
    fullspec = lambda: pl.BlockSpec((None,) * 0)
    out = pl.pallas_call(
        _enc_body,
        grid=(_G,),
        in_specs=[
            pl.BlockSpec(memory_space=pltpu.SMEM),
            pl.BlockSpec((_B, 128), lambda g: (g, 0)),
            pl.BlockSpec((128, H), lambda g: (0, 0)),
            pl.BlockSpec((1, H), lambda g: (0, 0)),
            pl.BlockSpec((128, H), lambda g: (0, 0)),
            pl.BlockSpec((1, H), lambda g: (0, 0)),
        ],
        out_specs=[
            pl.BlockSpec((_B, H), lambda g: (g, 0)),
            pl.BlockSpec((_B, H), lambda g: (g, 0)),
            pl.BlockSpec((_B, H), lambda g: (g, 0)),
            pl.BlockSpec((_B, H), lambda g: (g, 0)),
            pl.BlockSpec((1, H), lambda g: (0, 0)),
            pl.BlockSpec((1, H), lambda g: (0, 0)),
        ],
        out_shape=[
            jax.ShapeDtypeStruct((N, H), jnp.float32),
            jax.ShapeDtypeStruct((N, H), jnp.float32),
            jax.ShapeDtypeStruct((N, H), jnp.float32),
            jax.ShapeDtypeStruct((N, H), jnp.float32),
            jax.ShapeDtypeStruct((1, H), jnp.float32),
            jax.ShapeDtypeStruct((1, H), jnp.float32),
        ],
        scratch_shapes=[pltpu.VMEM((1, H), jnp.float32),
                        pltpu.VMEM((1, H), jnp.float32)],
    )(t0.reshape(1), xin, Wn, bn.reshape(1, H), We, be.reshape(1, H))
    return out


def _conv_tail(aux, Sn, bnd, z, row0, W1i, b1i, g1i, be1i, W2i, b2i):
    St = Sn[0] + Sn[1]
    bv = jnp.concatenate([bnd[0], bnd[1]], axis=0).reshape(2 * _NS, 2, 128)
    bacc = bv[:, 0, :]
    bids = bv[:, 1, 0:1].astype(jnp.int32)
    onehot = (bids == row0 +
              lax.broadcasted_iota(jnp.int32, (2 * _NS, _B), 1)
              ).astype(jnp.float32)
    St = St + lax.dot_general(onehot, bacc, (((0,), (0,)), ((), ())),
                              preferred_element_type=jnp.float32)
    s1 = St[:, :H]
    s2 = St[:, H:]
    agg = s2 / (s1 + 1e-30)
    nrm = jnp.maximum(
        jnp.sqrt(jnp.sum(agg * agg, axis=-1, keepdims=True)), 1e-12)
    xn = jnp.sqrt(jnp.sum(z * z, axis=-1, keepdims=True))
    out = (agg / nrm) * xn * aux[0] + z
    hmid = _ln(out @ W1i + b1i, g1i, be1i)
    return jax.nn.relu(hmid) @ W2i + b2i


def _mid_body(aux, h, z, Sn, bnd, p, cnpd, W1i, b1i, g1i, be1i, W2i, b2i,
              glnn, blnn, hn, zn, qn, mtn, accq, *, first):
    g = pl.program_id(0)
    hd = _conv_tail(aux, Sn[...], bnd[...], z[...], g * _B, W1i[...],
                    b1i[...], g1i[...], be1i[...], W2i[...], b2i[...])
    hnew = hd if first else h[...] + hd
    znew = jax.nn.relu(_ln(hnew, glnn[...], blnn[...]))
    qnew = znew + p[...]
    hn[...] = hnew
    zn[...] = znew
    qn[...] = qnew
    _colmax_update(accq, jnp.max(qnew, axis=0, keepdims=True), g)

    @pl.when(g == _G - 1)
    def _():
        tn = aux[1]
        bmsg = jnp.maximum(accq[...] + cnpd[...], 0.0) + 1e-7
        mtn[...] = jnp.maximum(tn * bmsg, tn * 1e-7)


_vec = lambda r: pl.BlockSpec((1, r), lambda g: (0, 0))


def _mid(h, z, S, p, cnpd, Wb, glnn, blnn, scale_i, t_next, first):
    W1i, b1i, g1i, be1i, W2i, b2i = Wb
    aux = jnp.stack([scale_i, t_next])
    nspec = lambda: pl.BlockSpec((_B, H), lambda g: (g, 0))
    out = pl.pallas_call(
        functools.partial(_mid_body, first=first),
        grid=(_G,),
        in_specs=[
            pl.BlockSpec(memory_space=pltpu.SMEM),
            nspec(), nspec(),
            pl.BlockSpec((_NC, _B, 128), lambda g: (0, g, 0)),
            pl.BlockSpec((_NC, 2 * _NS, 128), lambda g: (0, 0, 0)),
            nspec(), _vec(H),
            pl.BlockSpec((H, 2 * H), lambda g: (0, 0)), _vec(2 * H),
            _vec(2 * H), _vec(2 * H),
            pl.BlockSpec((2 * H, H), lambda g: (0, 0)), _vec(H),
            _vec(H), _vec(H),
        ],
        out_specs=[nspec(), nspec(), nspec(), _vec(H)],
        out_shape=[
            jax.ShapeDtypeStruct((N, H), jnp.float32),
            jax.ShapeDtypeStruct((N, H), jnp.float32),
            jax.ShapeDtypeStruct((N, H), jnp.float32),
            jax.ShapeDtypeStruct((1, H), jnp.float32),
        ],
        scratch_shapes=[pltpu.VMEM((1, H), jnp.float32)],
    )(aux, h, z, S[:, :N], S[:, N:], p, cnpd, W1i, b1i.reshape(1, -1),
      g1i.reshape(1, -1), be1i.reshape(1, -1), W2i, b2i.reshape(1, -1),
      glnn.reshape(1, H), blnn.reshape(1, H))
    return out


def _last_body(aux, h, z, Sn, bnd, W1i, b1i, g1i, be1i, W2i, b2i, gln0,
               bln0, Wlin, blin, y):
    g = pl.program_id(0)
    hd = _conv_tail(aux, Sn[...], bnd[...], z[...], g * _B, W1i[...],
                    b1i[...], g1i[...], be1i[...], W2i[...], b2i[...])
    hnew = h[...] + hd
    hf = jax.nn.relu(_ln(hnew, gln0[...], bln0[...]))
    y[...] = hf @ Wlin[...] + blin[...]


def _last(h, z, S, Wb, gln0, bln0, Wlin, blin, scale_i):
    W1i, b1i, g1i, be1i, W2i, b2i = Wb
    aux = jnp.stack([scale_i, scale_i])
    nspec = lambda: pl.BlockSpec((_B, H), lambda g: (g, 0))
    OUT = Wlin.shape[1]
    return pl.pallas_call(
        _last_body,
        grid=(_G,),
        in_specs=[
            pl.BlockSpec(memory_space=pltpu.SMEM),
            nspec(), nspec(),
            pl.BlockSpec((_NC, _B, 128), lambda g: (0, g, 0)),
            pl.BlockSpec((_NC, 2 * _NS, 128), lambda g: (0, 0, 0)),
            pl.BlockSpec((H, 2 * H), lambda g: (0, 0)), _vec(2 * H),
            _vec(2 * H), _vec(2 * H),
            pl.BlockSpec((2 * H, H), lambda g: (0, 0)), _vec(H),
            _vec(H), _vec(H),
            pl.BlockSpec((H, OUT), lambda g: (0, 0)), _vec(OUT),
        ],
        out_specs=pl.BlockSpec((_B, OUT), lambda g: (g, 0)),
        out_shape=jax.ShapeDtypeStruct((N, OUT), jnp.float32),
    )(aux, h, z, S[:, :N], S[:, N:], W1i, b1i.reshape(1, -1),
      g1i.reshape(1, -1), be1i.reshape(1, -1), W2i, b2i.reshape(1, -1),
      gln0.reshape(1, H), bln0.reshape(1, H), Wlin, blin.reshape(1, -1))


def kernel(x, pos, edge_index, Wn, bn, We, be, t, scale, W1, b1, g1, be1,
           W2, b2, gln, bln, Wlin, blin):
    xin = jnp.concatenate([x, jax.lax.stop_gradient(pos)], axis=-1)
    src = edge_index[1]
    dst = edge_index[0]
    perm = jnp.argsort(dst)
    ssrc = src[perm]
    sdst = dst[perm]
    idx2 = jnp.stack([ssrc, sdst], axis=1).reshape(-1)

    h, q, p, npd, mt, cnpd = _encoder(xin, Wn, bn, We, be, t[0])
    z = h

    for i in range(L):
        tbl = jnp.concatenate([q, npd], axis=1)
        tv = jnp.broadcast_to(t[i], (16,)).astype(jnp.float32)
        S = _sc_edges(tbl, idx2, mt.reshape(H), tv)
        Wb = (W1[i], b1[i], g1[i], be1[i], W2[i], b2[i])
        if i < L - 1:
            h, z, q, mt = _mid(h, z, S, p, cnpd, Wb, gln[i + 1], bln[i + 1],
                               scale[i], t[i + 1], first=(i == 0))
        else:
            y = _last(h, z, S, Wb, gln[0], bln[0], Wlin, blin, scale[i])
    return y
